# Initial kernel scaffold; baseline (speedup 1.0000x reference)
import functools

import jax
import jax.numpy as jnp
from jax import lax
from jax.experimental import pallas as pl
from jax.experimental.pallas import tpu as pltpu
from jax.experimental.pallas import tpu_sc as plsc

_N = 10000
_E = 320000

_NC = 2
_NS = 16
_NW = _NC * _NS
_BLK = 80
_EPT = _E // _NW
_NBT = _EPT // _BLK
_RPT = _N // _NS


def _mesh():
  return plsc.VectorSubcoreMesh(
      core_axis_name="c", subcore_axis_name="s",
      num_cores=_NC, num_subcores=_NS)


def _make_sc_agg(wc):
  @functools.partial(
      pl.kernel,
      out_type=(jax.ShapeDtypeStruct((_N, wc), jnp.float32),
                jax.ShapeDtypeStruct((_N, wc), jnp.float32)),
      mesh=_mesh(),
      scratch_types=[
          pltpu.VMEM((_NBT, _BLK), jnp.int32),
          pltpu.VMEM((_NBT, _BLK), jnp.int32),
          pltpu.VMEM((_BLK, wc), jnp.float32),
          pltpu.VMEM_SHARED((_N, wc), jnp.float32),
          pltpu.SemaphoreType.DMA,
      ])
  def k(table_h, src_h, dst_h, zeros_h, out0, out1, src_v, dst_v, rows_v,
        acc, sem):
    co = lax.axis_index("c")
    sid = lax.axis_index("s")
    wid = co * _NS + sid
    rs = sid * _RPT

    pltpu.sync_copy(zeros_h.at[pl.ds(rs, _RPT)], acc.at[pl.ds(rs, _RPT)])

    pltpu.sync_copy(src_h.at[pl.ds(wid * _NBT, _NBT)], src_v)
    pltpu.sync_copy(dst_h.at[pl.ds(wid * _NBT, _NBT)], dst_v)
    plsc.subcore_barrier()

    def body(b, carry):
      pltpu.async_copy(table_h.at[src_v.at[b]], rows_v, sem).wait()
      pltpu.sync_copy(rows_v, acc.at[dst_v.at[b]], add=True)
      return carry

    lax.fori_loop(0, _NBT, body, 0)
    plsc.subcore_barrier()

    @pl.when(co == 0)
    def _():
      pltpu.sync_copy(acc.at[pl.ds(rs, _RPT)], out0.at[pl.ds(rs, _RPT)])

    @pl.when(co == 1)
    def _():
      pltpu.sync_copy(acc.at[pl.ds(rs, _RPT)], out1.at[pl.ds(rs, _RPT)])

  return k


def _make_sc_deg():
  wc = 16

  @functools.partial(
      pl.kernel,
      out_type=(jax.ShapeDtypeStruct((_N, wc), jnp.float32),
                jax.ShapeDtypeStruct((_N, wc), jnp.float32)),
      mesh=_mesh(),
      scratch_types=[
          pltpu.VMEM((_NBT, _BLK), jnp.int32),
          pltpu.VMEM((_BLK, wc), jnp.float32),
          pltpu.VMEM_SHARED((_N, wc), jnp.float32),
      ])
  def k(dst_h, ones_h, zeros_h, out0, out1, dst_v, ones_v, acc):
    co = lax.axis_index("c")
    sid = lax.axis_index("s")
    wid = co * _NS + sid
    rs = sid * _RPT

    pltpu.sync_copy(zeros_h.at[pl.ds(rs, _RPT)], acc.at[pl.ds(rs, _RPT)])
    pltpu.sync_copy(ones_h, ones_v)
    pltpu.sync_copy(dst_h.at[pl.ds(wid * _NBT, _NBT)], dst_v)
    plsc.subcore_barrier()

    def body(b, carry):
      pltpu.sync_copy(ones_v, acc.at[dst_v.at[b]], add=True)
      return carry

    lax.fori_loop(0, _NBT, body, 0)
    plsc.subcore_barrier()

    @pl.when(co == 0)
    def _():
      pltpu.sync_copy(acc.at[pl.ds(rs, _RPT)], out0.at[pl.ds(rs, _RPT)])

    @pl.when(co == 1)
    def _():
      pltpu.sync_copy(acc.at[pl.ds(rs, _RPT)], out1.at[pl.ds(rs, _RPT)])

  return k


_RB = 400


def _row_spec(d):
  return pl.BlockSpec((_RB, d), lambda i: (i, 0))


def _full_spec(a, b):
  return pl.BlockSpec((a, b), lambda i: (0, 0))


def _tc_layer0(x, a0, a1, d0, d1, Wl0, bl0, Wr0, Wl1):
  dout = Wl0.shape[1]
  wn = Wl1.shape[1] // 2

  def body(x_r, a0_r, a1_r, d0_r, d1_r, Wl0_r, bl0_r, Wr0_r, Wl1_r,
           h1_o, u0_o, u1_o, invd_o):
    deg = jnp.maximum(d0_r[:, 0:1] + d1_r[:, 0:1], 1.0)
    invd = 1.0 / deg
    invd_o[...] = invd
    mean = (a0_r[...] + a1_r[...]) * invd
    h1 = jnp.tanh(jnp.dot(mean, Wl0_r[...], preferred_element_type=jnp.float32)
                  + bl0_r[...] +
                  jnp.dot(x_r[...], Wr0_r[...],
                          preferred_element_type=jnp.float32))
    h1_o[...] = h1
    u1 = jnp.dot(h1, Wl1_r[...], preferred_element_type=jnp.float32)
    u0_o[...] = u1[:, :wn]
    u1_o[...] = u1[:, wn:]

  din = x.shape[1]
  return pl.pallas_call(
      body,
      grid=(_N // _RB,),
      in_specs=[
          _row_spec(din), _row_spec(din), _row_spec(din),
          _row_spec(16), _row_spec(16),
          _full_spec(din, dout), pl.BlockSpec((dout,), lambda i: (0,)),
          _full_spec(din, dout), _full_spec(dout, 2 * wn),
      ],
      out_specs=[_row_spec(dout), _row_spec(wn), _row_spec(wn), _row_spec(1)],
      out_shape=[
          jax.ShapeDtypeStruct((_N, dout), jnp.float32),
          jax.ShapeDtypeStruct((_N, wn), jnp.float32),
          jax.ShapeDtypeStruct((_N, wn), jnp.float32),
          jax.ShapeDtypeStruct((_N, 1), jnp.float32),
      ],
  )(x, a0, a1, d0, d1, Wl0, bl0, Wr0, Wl1)


def _tc_layer_mid(h, parts, invd, bl, Wr, Wl_next, nsplit):
  din = h.shape[1]
  dout = bl.shape[0]
  nch = len(parts) // 2
  wc = dout // nch
  dn = Wl_next.shape[1]
  wn = dn // nsplit

  def body(*refs):
    h_r = refs[0]
    part_r = refs[1:1 + 2 * nch]
    invd_r, bl_r, Wr_r, Wln_r = refs[1 + 2 * nch:5 + 2 * nch]
    outs = refs[5 + 2 * nch:]
    hn_o = outs[0]
    u_o = outs[1:]
    invd = invd_r[...]
    mean = jnp.concatenate(
        [(part_r[2 * j][...] + part_r[2 * j + 1][...]) * invd
         for j in range(nch)], axis=1)
    hn = jnp.tanh(mean + bl_r[...] +
                  jnp.dot(h_r[...], Wr_r[...],
                          preferred_element_type=jnp.float32))
    hn_o[...] = hn
    un = jnp.dot(hn, Wln_r[...], preferred_element_type=jnp.float32)
    for s in range(nsplit):
      u_o[s][...] = un[:, s * wn:(s + 1) * wn]

  in_specs = ([_row_spec(din)] + [_row_spec(wc)] * (2 * nch) +
              [_row_spec(1), pl.BlockSpec((dout,), lambda i: (0,)),
               _full_spec(din, dout), _full_spec(dout, dn)])
  out_specs = [_row_spec(dout)] + [_row_spec(wn)] * nsplit
  out_shape = ([jax.ShapeDtypeStruct((_N, dout), jnp.float32)] +
               [jax.ShapeDtypeStruct((_N, wn), jnp.float32)] * nsplit)
  return pl.pallas_call(
      body, grid=(_N // _RB,),
      in_specs=in_specs, out_specs=out_specs, out_shape=out_shape,
  )(h, *parts, invd, bl, Wr, Wl_next)


def _tc_layer_last(h, parts, invd, bl, Wr):
  din = h.shape[1]
  dout = bl.shape[0]

  def body(h_r, p0_r, p1_r, invd_r, bl_r, Wr_r, out_o):
    mean = (p0_r[...] + p1_r[...]) * invd_r[...]
    out_o[...] = jax.nn.sigmoid(
        mean + bl_r[...] +
        jnp.dot(h_r[...], Wr_r[...], preferred_element_type=jnp.float32))

  return pl.pallas_call(
      body, grid=(_N // _RB,),
      in_specs=[_row_spec(din), _row_spec(dout), _row_spec(dout),
                _row_spec(1), pl.BlockSpec((dout,), lambda i: (0,)),
                _full_spec(din, dout)],
      out_specs=[_row_spec(dout)],
      out_shape=[jax.ShapeDtypeStruct((_N, dout), jnp.float32)],
  )(h, parts[0], parts[1], invd, bl, Wr)[0]


def kernel(x, edge_index, batch, Wl0, bl0, Wr0, Wl1, bl1, Wr1, Wl2, bl2, Wr2,
           Wl3, bl3, Wr3, Wl4, bl4, Wr4):
  src2 = edge_index[0].reshape(_E // _BLK, _BLK)
  dst2 = edge_index[1].reshape(_E // _BLK, _BLK)

  ones16 = jnp.ones((_BLK, 16), jnp.float32)
  zeros16 = jnp.zeros((_N, 16), jnp.float32)
  deg0, deg1 = _make_sc_deg()(dst2, ones16, zeros16)

  z128 = jnp.zeros((_N, 128), jnp.float32)
  a0, a1 = _make_sc_agg(128)(x, src2, dst2, z128)
  h1, u1a, u1b, invd = _tc_layer0(x, a0, a1, deg0, deg1, Wl0, bl0, Wr0, Wl1)

  h = h1
  u_chunks = [u1a, u1b]
  mids = [(bl1, Wr1, Wl2), (bl2, Wr2, Wl3), (bl3, Wr3, Wl4)]
  nsplits = [2, 2, 1]
  for (bl, Wr, Wln), nsplit in zip(mids, nsplits):
    wc = u_chunks[0].shape[1]
    zc = jnp.zeros((_N, wc), jnp.float32)
    parts = []
    for u in u_chunks:
      p0, p1 = _make_sc_agg(wc)(u, src2, dst2, zc)
      parts += [p0, p1]
    outs = _tc_layer_mid(h, parts, invd, bl, Wr, Wln, nsplit)
    h = outs[0]
    u_chunks = list(outs[1:])

  z16 = jnp.zeros((_N, 16), jnp.float32)
  p0, p1 = _make_sc_agg(16)(u_chunks[0], src2, dst2, z16)
  return _tc_layer_last(h, [p0, p1], invd, bl4, Wr4)


# speedup vs baseline: 5.6939x; 5.6939x over previous
import functools

import jax
import jax.numpy as jnp
from jax import lax
from jax.experimental import pallas as pl
from jax.experimental.pallas import tpu as pltpu
from jax.experimental.pallas import tpu_sc as plsc

_N = 10000
_E = 320000

_NC = 2
_NS = 16
_NW = _NC * _NS
_BLK = 80
_EPT = _E // _NW
_NBT = _EPT // _BLK
_RPT = _N // _NS


def _mesh():
  return plsc.VectorSubcoreMesh(
      core_axis_name="c", subcore_axis_name="s",
      num_cores=_NC, num_subcores=_NS)


def _make_sc_agg(wc):
  @functools.partial(
      pl.kernel,
      out_type=(jax.ShapeDtypeStruct((_N, wc), jnp.float32),
                jax.ShapeDtypeStruct((_N, wc), jnp.float32)),
      mesh=_mesh(),
      scratch_types=[
          pltpu.VMEM((_NBT, _BLK), jnp.int32),
          pltpu.VMEM((_NBT, _BLK), jnp.int32),
          pltpu.VMEM((_BLK, wc), jnp.float32),
          pltpu.VMEM_SHARED((_N, wc), jnp.float32),
          pltpu.SemaphoreType.DMA,
      ],
      compiler_params=pltpu.CompilerParams(use_tc_tiling_on_sc=False))
  def k(table_h, src_h, dst_h, zeros_h, out0, out1, src_v, dst_v, rows_v,
        acc, sem):
    co = lax.axis_index("c")
    sid = lax.axis_index("s")
    wid = co * _NS + sid
    rs = sid * _RPT

    pltpu.sync_copy(zeros_h.at[pl.ds(rs, _RPT)], acc.at[pl.ds(rs, _RPT)])

    pltpu.sync_copy(src_h.at[pl.ds(wid * _NBT, _NBT)], src_v)
    pltpu.sync_copy(dst_h.at[pl.ds(wid * _NBT, _NBT)], dst_v)
    plsc.subcore_barrier()

    def body(b, carry):
      pltpu.async_copy(table_h.at[src_v.at[b]], rows_v, sem).wait()
      pltpu.sync_copy(rows_v, acc.at[dst_v.at[b]], add=True)
      return carry

    lax.fori_loop(0, _NBT, body, 0)
    plsc.subcore_barrier()

    @pl.when(co == 0)
    def _():
      pltpu.sync_copy(acc.at[pl.ds(rs, _RPT)], out0.at[pl.ds(rs, _RPT)])

    @pl.when(co == 1)
    def _():
      pltpu.sync_copy(acc.at[pl.ds(rs, _RPT)], out1.at[pl.ds(rs, _RPT)])

  return k


def _make_sc_deg():
  wc = 16

  @functools.partial(
      pl.kernel,
      out_type=(jax.ShapeDtypeStruct((_N, wc), jnp.float32),
                jax.ShapeDtypeStruct((_N, wc), jnp.float32)),
      mesh=_mesh(),
      scratch_types=[
          pltpu.VMEM((_NBT, _BLK), jnp.int32),
          pltpu.VMEM((_BLK, wc), jnp.float32),
          pltpu.VMEM_SHARED((_N, wc), jnp.float32),
      ],
      compiler_params=pltpu.CompilerParams(use_tc_tiling_on_sc=False))
  def k(dst_h, ones_h, zeros_h, out0, out1, dst_v, ones_v, acc):
    co = lax.axis_index("c")
    sid = lax.axis_index("s")
    wid = co * _NS + sid
    rs = sid * _RPT

    pltpu.sync_copy(zeros_h.at[pl.ds(rs, _RPT)], acc.at[pl.ds(rs, _RPT)])
    pltpu.sync_copy(ones_h, ones_v)
    pltpu.sync_copy(dst_h.at[pl.ds(wid * _NBT, _NBT)], dst_v)
    plsc.subcore_barrier()

    def body(b, carry):
      pltpu.sync_copy(ones_v, acc.at[dst_v.at[b]], add=True)
      return carry

    lax.fori_loop(0, _NBT, body, 0)
    plsc.subcore_barrier()

    @pl.when(co == 0)
    def _():
      pltpu.sync_copy(acc.at[pl.ds(rs, _RPT)], out0.at[pl.ds(rs, _RPT)])

    @pl.when(co == 1)
    def _():
      pltpu.sync_copy(acc.at[pl.ds(rs, _RPT)], out1.at[pl.ds(rs, _RPT)])

  return k


_RB = 400


def _row_spec(d):
  return pl.BlockSpec((_RB, d), lambda i: (i, 0))


def _full_spec(a, b):
  return pl.BlockSpec((a, b), lambda i: (0, 0))


def _tc_layer0(x, a0, a1, d0, d1, Wl0, bl0, Wr0, Wl1):
  dout = Wl0.shape[1]
  wn = Wl1.shape[1] // 2

  def body(x_r, a0_r, a1_r, d0_r, d1_r, Wl0_r, bl0_r, Wr0_r, Wl1_r,
           h1_o, u0_o, u1_o, invd_o):
    deg = jnp.maximum(d0_r[:, 0:1] + d1_r[:, 0:1], 1.0)
    invd = 1.0 / deg
    invd_o[...] = invd
    mean = (a0_r[...] + a1_r[...]) * invd
    h1 = jnp.tanh(jnp.dot(mean, Wl0_r[...], preferred_element_type=jnp.float32)
                  + bl0_r[0, :] +
                  jnp.dot(x_r[...], Wr0_r[...],
                          preferred_element_type=jnp.float32))
    h1_o[...] = h1
    u1 = jnp.dot(h1, Wl1_r[...], preferred_element_type=jnp.float32)
    u0_o[...] = u1[:, :wn]
    u1_o[...] = u1[:, wn:]

  din = x.shape[1]
  return pl.pallas_call(
      body,
      grid=(_N // _RB,),
      in_specs=[
          _row_spec(din), _row_spec(din), _row_spec(din),
          _row_spec(16), _row_spec(16),
          _full_spec(din, dout), _full_spec(1, dout),
          _full_spec(din, dout), _full_spec(dout, 2 * wn),
      ],
      out_specs=[_row_spec(dout), _row_spec(wn), _row_spec(wn), _row_spec(1)],
      out_shape=[
          jax.ShapeDtypeStruct((_N, dout), jnp.float32),
          jax.ShapeDtypeStruct((_N, wn), jnp.float32),
          jax.ShapeDtypeStruct((_N, wn), jnp.float32),
          jax.ShapeDtypeStruct((_N, 1), jnp.float32),
      ],
  )(x, a0, a1, d0, d1, Wl0, bl0, Wr0, Wl1)


def _tc_layer_mid(h, parts, invd, bl, Wr, Wl_next, nsplit):
  din = h.shape[1]
  dout = bl.shape[1]
  nch = len(parts) // 2
  wc = dout // nch
  dn = Wl_next.shape[1]
  wn = dn // nsplit

  def body(*refs):
    h_r = refs[0]
    part_r = refs[1:1 + 2 * nch]
    invd_r, bl_r, Wr_r, Wln_r = refs[1 + 2 * nch:5 + 2 * nch]
    outs = refs[5 + 2 * nch:]
    hn_o = outs[0]
    u_o = outs[1:]
    invd = invd_r[...]
    mean = jnp.concatenate(
        [(part_r[2 * j][...] + part_r[2 * j + 1][...]) * invd
         for j in range(nch)], axis=1)
    hn = jnp.tanh(mean + bl_r[0, :] +
                  jnp.dot(h_r[...], Wr_r[...],
                          preferred_element_type=jnp.float32))
    hn_o[...] = hn
    un = jnp.dot(hn, Wln_r[...], preferred_element_type=jnp.float32)
    for s in range(nsplit):
      u_o[s][...] = un[:, s * wn:(s + 1) * wn]

  in_specs = ([_row_spec(din)] + [_row_spec(wc)] * (2 * nch) +
              [_row_spec(1), _full_spec(1, dout),
               _full_spec(din, dout), _full_spec(dout, dn)])
  out_specs = [_row_spec(dout)] + [_row_spec(wn)] * nsplit
  out_shape = ([jax.ShapeDtypeStruct((_N, dout), jnp.float32)] +
               [jax.ShapeDtypeStruct((_N, wn), jnp.float32)] * nsplit)
  return pl.pallas_call(
      body, grid=(_N // _RB,),
      in_specs=in_specs, out_specs=out_specs, out_shape=out_shape,
  )(h, *parts, invd, bl, Wr, Wl_next)


def _tc_layer_last(h, parts, invd, bl, Wr):
  din = h.shape[1]
  dout = bl.shape[1]

  def body(h_r, p0_r, p1_r, invd_r, bl_r, Wr_r, out_o):
    mean = (p0_r[...] + p1_r[...]) * invd_r[...]
    out_o[...] = jax.nn.sigmoid(
        mean + bl_r[0, :] +
        jnp.dot(h_r[...], Wr_r[...], preferred_element_type=jnp.float32))

  return pl.pallas_call(
      body, grid=(_N // _RB,),
      in_specs=[_row_spec(din), _row_spec(dout), _row_spec(dout),
                _row_spec(1), _full_spec(1, dout),
                _full_spec(din, dout)],
      out_specs=[_row_spec(dout)],
      out_shape=[jax.ShapeDtypeStruct((_N, dout), jnp.float32)],
  )(h, parts[0], parts[1], invd, bl, Wr)[0]


def kernel(x, edge_index, batch, Wl0, bl0, Wr0, Wl1, bl1, Wr1, Wl2, bl2, Wr2,
           Wl3, bl3, Wr3, Wl4, bl4, Wr4):
  bl0, bl1, bl2, bl3, bl4 = (b.reshape(1, -1) for b in (bl0, bl1, bl2, bl3, bl4))
  src2 = edge_index[0].reshape(_E // _BLK, _BLK)
  dst2 = edge_index[1].reshape(_E // _BLK, _BLK)

  ones16 = jnp.ones((_BLK, 16), jnp.float32)
  zeros16 = jnp.zeros((_N, 16), jnp.float32)
  deg0, deg1 = _make_sc_deg()(dst2, ones16, zeros16)

  z128 = jnp.zeros((_N, 128), jnp.float32)
  a0, a1 = _make_sc_agg(128)(x, src2, dst2, z128)
  h1, u1a, u1b, invd = _tc_layer0(x, a0, a1, deg0, deg1, Wl0, bl0, Wr0, Wl1)

  h = h1
  u_chunks = [u1a, u1b]
  mids = [(bl1, Wr1, Wl2), (bl2, Wr2, Wl3), (bl3, Wr3, Wl4)]
  nsplits = [2, 2, 1]
  for (bl, Wr, Wln), nsplit in zip(mids, nsplits):
    wc = u_chunks[0].shape[1]
    zc = jnp.zeros((_N, wc), jnp.float32)
    parts = []
    for u in u_chunks:
      p0, p1 = _make_sc_agg(wc)(u, src2, dst2, zc)
      parts += [p0, p1]
    outs = _tc_layer_mid(h, parts, invd, bl, Wr, Wln, nsplit)
    h = outs[0]
    u_chunks = list(outs[1:])

  z16 = jnp.zeros((_N, 16), jnp.float32)
  p0, p1 = _make_sc_agg(16)(u_chunks[0], src2, dst2, z16)
  return _tc_layer_last(h, [p0, p1], invd, bl4, Wr4)


# speedup vs baseline: 5.9802x; 1.0503x over previous
import functools

import jax
import jax.numpy as jnp
from jax import lax
from jax.experimental import pallas as pl
from jax.experimental.pallas import tpu as pltpu
from jax.experimental.pallas import tpu_sc as plsc

_N = 10000
_E = 320000

_NC = 2
_NS = 16
_NW = _NC * _NS
_BLK = 80
_EPT = _E // _NW
_NBT = _EPT // _BLK
_RPT = _N // _NS
_K = 5


def _mesh():
  return plsc.VectorSubcoreMesh(
      core_axis_name="c", subcore_axis_name="s",
      num_cores=_NC, num_subcores=_NS)


def _make_sc_agg(wc, nch):
  @functools.partial(
      pl.kernel,
      out_type=tuple(jax.ShapeDtypeStruct((_N, wc), jnp.float32)
                     for _ in range(2 * nch)),
      mesh=_mesh(),
      scratch_types=[
          pltpu.VMEM((_NBT, _BLK), jnp.int32),
          pltpu.VMEM((_NBT, _BLK), jnp.int32),
          pltpu.VMEM((_K, _BLK, wc), jnp.float32),
          pltpu.VMEM_SHARED((_N, wc), jnp.float32),
          pltpu.SemaphoreType.DMA,
          pltpu.SemaphoreType.DMA,
      ],
      compiler_params=pltpu.CompilerParams(use_tc_tiling_on_sc=False))
  def k(*refs):
    tables = refs[:nch]
    src_h, dst_h, zeros_h = refs[nch:nch + 3]
    outs = refs[nch + 3:nch + 3 + 2 * nch]
    src_v, dst_v, rows_v, acc, gsem, ssem = refs[nch + 3 + 2 * nch:]
    co = lax.axis_index("c")
    sid = lax.axis_index("s")
    wid = co * _NS + sid
    rs = sid * _RPT

    pltpu.sync_copy(src_h.at[pl.ds(wid * _NBT, _NBT)], src_v)
    pltpu.sync_copy(dst_h.at[pl.ds(wid * _NBT, _NBT)], dst_v)

    for j in range(nch):
      table_h = tables[j]
      pltpu.sync_copy(zeros_h.at[pl.ds(rs, _RPT)], acc.at[pl.ds(rs, _RPT)])
      plsc.subcore_barrier()

      def body(o, carry):
        b0 = o * _K
        gd = [pltpu.async_copy(table_h.at[src_v.at[b0 + p]], rows_v.at[p],
                               gsem)
              for p in range(_K)]
        for d in gd:
          d.wait()
        sd = [pltpu.async_copy(rows_v.at[p], acc.at[dst_v.at[b0 + p]], ssem,
                               add=True)
              for p in range(_K)]
        for d in sd:
          d.wait()
        return carry

      lax.fori_loop(0, _NBT // _K, body, 0)
      plsc.subcore_barrier()

      @pl.when(co == 0)
      def _():
        pltpu.sync_copy(acc.at[pl.ds(rs, _RPT)],
                        outs[2 * j].at[pl.ds(rs, _RPT)])

      @pl.when(co == 1)
      def _():
        pltpu.sync_copy(acc.at[pl.ds(rs, _RPT)],
                        outs[2 * j + 1].at[pl.ds(rs, _RPT)])

  return k


def _make_sc_deg():
  wc = 16

  @functools.partial(
      pl.kernel,
      out_type=(jax.ShapeDtypeStruct((_N, wc), jnp.float32),
                jax.ShapeDtypeStruct((_N, wc), jnp.float32)),
      mesh=_mesh(),
      scratch_types=[
          pltpu.VMEM((_NBT, _BLK), jnp.int32),
          pltpu.VMEM((_BLK, wc), jnp.float32),
          pltpu.VMEM_SHARED((_N, wc), jnp.float32),
          pltpu.SemaphoreType.DMA,
      ],
      compiler_params=pltpu.CompilerParams(use_tc_tiling_on_sc=False))
  def k(dst_h, ones_h, zeros_h, out0, out1, dst_v, ones_v, acc, ssem):
    co = lax.axis_index("c")
    sid = lax.axis_index("s")
    wid = co * _NS + sid
    rs = sid * _RPT

    pltpu.sync_copy(zeros_h.at[pl.ds(rs, _RPT)], acc.at[pl.ds(rs, _RPT)])
    pltpu.sync_copy(ones_h, ones_v)
    pltpu.sync_copy(dst_h.at[pl.ds(wid * _NBT, _NBT)], dst_v)
    plsc.subcore_barrier()

    def body(o, carry):
      b0 = o * _K
      sd = [pltpu.async_copy(ones_v, acc.at[dst_v.at[b0 + p]], ssem, add=True)
            for p in range(_K)]
      for d in sd:
        d.wait()
      return carry

    lax.fori_loop(0, _NBT // _K, body, 0)
    plsc.subcore_barrier()

    @pl.when(co == 0)
    def _():
      pltpu.sync_copy(acc.at[pl.ds(rs, _RPT)], out0.at[pl.ds(rs, _RPT)])

    @pl.when(co == 1)
    def _():
      pltpu.sync_copy(acc.at[pl.ds(rs, _RPT)], out1.at[pl.ds(rs, _RPT)])

  return k


_RB = 400


def _row_spec(d):
  return pl.BlockSpec((_RB, d), lambda i: (i, 0))


def _full_spec(a, b):
  return pl.BlockSpec((a, b), lambda i: (0, 0))


def _tc_layer0(x, parts, d0, d1, Wl0, bl0, Wr0, Wl1, nsplit):
  dout = Wl0.shape[1]
  din = x.shape[1]
  nch = len(parts) // 2
  wc = din // nch
  dn = Wl1.shape[1]
  wn = dn // nsplit

  def body(*refs):
    x_r = refs[0]
    part_r = refs[1:1 + 2 * nch]
    d0_r, d1_r, Wl0_r, bl0_r, Wr0_r, Wl1_r = refs[1 + 2 * nch:7 + 2 * nch]
    outs = refs[7 + 2 * nch:]
    h1_o, invd_o = outs[0], outs[1]
    u_o = outs[2:]
    deg = jnp.maximum(d0_r[:, 0:1] + d1_r[:, 0:1], 1.0)
    invd = 1.0 / deg
    invd_o[...] = invd
    mean = jnp.concatenate(
        [(part_r[2 * j][...] + part_r[2 * j + 1][...]) * invd
         for j in range(nch)], axis=1)
    h1 = jnp.tanh(jnp.dot(mean, Wl0_r[...], preferred_element_type=jnp.float32)
                  + bl0_r[0, :] +
                  jnp.dot(x_r[...], Wr0_r[...],
                          preferred_element_type=jnp.float32))
    h1_o[...] = h1
    u1 = jnp.dot(h1, Wl1_r[...], preferred_element_type=jnp.float32)
    for s in range(nsplit):
      u_o[s][...] = u1[:, s * wn:(s + 1) * wn]

  in_specs = ([_row_spec(din)] + [_row_spec(wc)] * (2 * nch) +
              [_row_spec(16), _row_spec(16),
               _full_spec(din, dout), _full_spec(1, dout),
               _full_spec(din, dout), _full_spec(dout, dn)])
  out_specs = ([_row_spec(dout), _row_spec(1)] + [_row_spec(wn)] * nsplit)
  out_shape = ([jax.ShapeDtypeStruct((_N, dout), jnp.float32),
                jax.ShapeDtypeStruct((_N, 1), jnp.float32)] +
               [jax.ShapeDtypeStruct((_N, wn), jnp.float32)] * nsplit)
  return pl.pallas_call(
      body, grid=(_N // _RB,),
      in_specs=in_specs, out_specs=out_specs, out_shape=out_shape,
  )(x, *parts, d0, d1, Wl0, bl0, Wr0, Wl1)


def _tc_layer_mid(h, parts, invd, bl, Wr, Wl_next, nsplit):
  din = h.shape[1]
  dout = bl.shape[1]
  nch = len(parts) // 2
  wc = dout // nch
  dn = Wl_next.shape[1]
  wn = dn // nsplit

  def body(*refs):
    h_r = refs[0]
    part_r = refs[1:1 + 2 * nch]
    invd_r, bl_r, Wr_r, Wln_r = refs[1 + 2 * nch:5 + 2 * nch]
    outs = refs[5 + 2 * nch:]
    hn_o = outs[0]
    u_o = outs[1:]
    invd = invd_r[...]
    mean = jnp.concatenate(
        [(part_r[2 * j][...] + part_r[2 * j + 1][...]) * invd
         for j in range(nch)], axis=1)
    hn = jnp.tanh(mean + bl_r[0, :] +
                  jnp.dot(h_r[...], Wr_r[...],
                          preferred_element_type=jnp.float32))
    hn_o[...] = hn
    un = jnp.dot(hn, Wln_r[...], preferred_element_type=jnp.float32)
    for s in range(nsplit):
      u_o[s][...] = un[:, s * wn:(s + 1) * wn]

  in_specs = ([_row_spec(din)] + [_row_spec(wc)] * (2 * nch) +
              [_row_spec(1), _full_spec(1, dout),
               _full_spec(din, dout), _full_spec(dout, dn)])
  out_specs = [_row_spec(dout)] + [_row_spec(wn)] * nsplit
  out_shape = ([jax.ShapeDtypeStruct((_N, dout), jnp.float32)] +
               [jax.ShapeDtypeStruct((_N, wn), jnp.float32)] * nsplit)
  return pl.pallas_call(
      body, grid=(_N // _RB,),
      in_specs=in_specs, out_specs=out_specs, out_shape=out_shape,
  )(h, *parts, invd, bl, Wr, Wl_next)


def _tc_layer_last(h, parts, invd, bl, Wr):
  din = h.shape[1]
  dout = bl.shape[1]

  def body(h_r, p0_r, p1_r, invd_r, bl_r, Wr_r, out_o):
    mean = (p0_r[...] + p1_r[...]) * invd_r[...]
    out_o[...] = jax.nn.sigmoid(
        mean + bl_r[0, :] +
        jnp.dot(h_r[...], Wr_r[...], preferred_element_type=jnp.float32))

  return pl.pallas_call(
      body, grid=(_N // _RB,),
      in_specs=[_row_spec(din), _row_spec(dout), _row_spec(dout),
                _row_spec(1), _full_spec(1, dout),
                _full_spec(din, dout)],
      out_specs=[_row_spec(dout)],
      out_shape=[jax.ShapeDtypeStruct((_N, dout), jnp.float32)],
  )(h, parts[0], parts[1], invd, bl, Wr)[0]


def kernel(x, edge_index, batch, Wl0, bl0, Wr0, Wl1, bl1, Wr1, Wl2, bl2, Wr2,
           Wl3, bl3, Wr3, Wl4, bl4, Wr4):
  bl0, bl1, bl2, bl3, bl4 = (b.reshape(1, -1) for b in (bl0, bl1, bl2, bl3, bl4))
  src2 = edge_index[0].reshape(_E // _BLK, _BLK)
  dst2 = edge_index[1].reshape(_E // _BLK, _BLK)

  ones16 = jnp.ones((_BLK, 16), jnp.float32)
  zeros16 = jnp.zeros((_N, 16), jnp.float32)
  deg0, deg1 = _make_sc_deg()(dst2, ones16, zeros16)

  wc0 = 32
  z32 = jnp.zeros((_N, wc0), jnp.float32)
  x_chunks = [x[:, j * wc0:(j + 1) * wc0] for j in range(128 // wc0)]
  parts0 = list(_make_sc_agg(wc0, len(x_chunks))(*x_chunks, src2, dst2, z32))
  outs0 = _tc_layer0(x, parts0, deg0, deg1, Wl0, bl0, Wr0, Wl1, 224 // wc0)
  h1, invd = outs0[0], outs0[1]

  h = h1
  u_chunks = list(outs0[2:])
  mids = [(bl1, Wr1, Wl2), (bl2, Wr2, Wl3), (bl3, Wr3, Wl4)]
  nsplits = [192 // wc0, 160 // wc0, 1]
  for (bl, Wr, Wln), nsplit in zip(mids, nsplits):
    wc = u_chunks[0].shape[1]
    zc = z32 if wc == wc0 else jnp.zeros((_N, wc), jnp.float32)
    parts = list(_make_sc_agg(wc, len(u_chunks))(*u_chunks, src2, dst2, zc))
    outs = _tc_layer_mid(h, parts, invd, bl, Wr, Wln, nsplit)
    h = outs[0]
    u_chunks = list(outs[1:])

  z16 = jnp.zeros((_N, 16), jnp.float32)
  p0, p1 = _make_sc_agg(16, 1)(u_chunks[0], src2, dst2, z16)
  return _tc_layer_last(h, [p0, p1], invd, bl4, Wr4)


# speedup vs baseline: 9.9951x; 1.6714x over previous
import functools

import jax
import jax.numpy as jnp
from jax import lax
from jax.experimental import pallas as pl
from jax.experimental.pallas import tpu as pltpu
from jax.experimental.pallas import tpu_sc as plsc

_N = 10000
_E = 320000

_NC = 2
_NS = 16
_NW = _NC * _NS
_BLK = 80
_EPT = _E // _NW
_NBT = _EPT // _BLK
_RPT = _N // _NS
_K = 5

_BF = jnp.bfloat16


def _mesh():
  return plsc.VectorSubcoreMesh(
      core_axis_name="c", subcore_axis_name="s",
      num_cores=_NC, num_subcores=_NS)


def _make_sc_agg(wc, nch):
  @functools.partial(
      pl.kernel,
      out_type=tuple(jax.ShapeDtypeStruct((_N, wc), _BF)
                     for _ in range(2 * nch)),
      mesh=_mesh(),
      scratch_types=[
          pltpu.VMEM((_NBT, _BLK), jnp.int32),
          pltpu.VMEM((_NBT, _BLK), jnp.int32),
          pltpu.VMEM((_K, _BLK, wc), _BF),
          pltpu.VMEM_SHARED((_N, wc), _BF),
          pltpu.SemaphoreType.DMA,
          pltpu.SemaphoreType.DMA,
      ],
      compiler_params=pltpu.CompilerParams(use_tc_tiling_on_sc=False))
  def k(*refs):
    tables = refs[:nch]
    src_h, dst_h, zeros_h = refs[nch:nch + 3]
    outs = refs[nch + 3:nch + 3 + 2 * nch]
    src_v, dst_v, rows_v, acc, gsem, ssem = refs[nch + 3 + 2 * nch:]
    co = lax.axis_index("c")
    sid = lax.axis_index("s")
    wid = co * _NS + sid
    rs = sid * _RPT

    pltpu.sync_copy(src_h.at[pl.ds(wid * _NBT, _NBT)], src_v)
    pltpu.sync_copy(dst_h.at[pl.ds(wid * _NBT, _NBT)], dst_v)

    for j in range(nch):
      table_h = tables[j]
      pltpu.sync_copy(zeros_h.at[pl.ds(rs, _RPT)], acc.at[pl.ds(rs, _RPT)])
      plsc.subcore_barrier()

      def body(o, carry):
        b0 = o * _K
        gd = [pltpu.async_copy(table_h.at[src_v.at[b0 + p]], rows_v.at[p],
                               gsem)
              for p in range(_K)]
        for d in gd:
          d.wait()
        sd = [pltpu.async_copy(rows_v.at[p], acc.at[dst_v.at[b0 + p]], ssem,
                               add=True)
              for p in range(_K)]
        for d in sd:
          d.wait()
        return carry

      lax.fori_loop(0, _NBT // _K, body, 0)
      plsc.subcore_barrier()

      @pl.when(co == 0)
      def _():
        pltpu.sync_copy(acc.at[pl.ds(rs, _RPT)],
                        outs[2 * j].at[pl.ds(rs, _RPT)])

      @pl.when(co == 1)
      def _():
        pltpu.sync_copy(acc.at[pl.ds(rs, _RPT)],
                        outs[2 * j + 1].at[pl.ds(rs, _RPT)])

  return k


def _make_sc_deg():
  wc = 16

  @functools.partial(
      pl.kernel,
      out_type=(jax.ShapeDtypeStruct((_N, wc), _BF),
                jax.ShapeDtypeStruct((_N, wc), _BF)),
      mesh=_mesh(),
      scratch_types=[
          pltpu.VMEM((_NBT, _BLK), jnp.int32),
          pltpu.VMEM((_BLK, wc), _BF),
          pltpu.VMEM_SHARED((_N, wc), _BF),
          pltpu.SemaphoreType.DMA,
      ],
      compiler_params=pltpu.CompilerParams(use_tc_tiling_on_sc=False))
  def k(dst_h, ones_h, zeros_h, out0, out1, dst_v, ones_v, acc, ssem):
    co = lax.axis_index("c")
    sid = lax.axis_index("s")
    wid = co * _NS + sid
    rs = sid * _RPT

    pltpu.sync_copy(zeros_h.at[pl.ds(rs, _RPT)], acc.at[pl.ds(rs, _RPT)])
    pltpu.sync_copy(ones_h, ones_v)
    pltpu.sync_copy(dst_h.at[pl.ds(wid * _NBT, _NBT)], dst_v)
    plsc.subcore_barrier()

    def body(o, carry):
      b0 = o * _K
      sd = [pltpu.async_copy(ones_v, acc.at[dst_v.at[b0 + p]], ssem, add=True)
            for p in range(_K)]
      for d in sd:
        d.wait()
      return carry

    lax.fori_loop(0, _NBT // _K, body, 0)
    plsc.subcore_barrier()

    @pl.when(co == 0)
    def _():
      pltpu.sync_copy(acc.at[pl.ds(rs, _RPT)], out0.at[pl.ds(rs, _RPT)])

    @pl.when(co == 1)
    def _():
      pltpu.sync_copy(acc.at[pl.ds(rs, _RPT)], out1.at[pl.ds(rs, _RPT)])

  return k


_RB = 400


def _row_spec(d):
  return pl.BlockSpec((_RB, d), lambda i: (i, 0))


def _full_spec(a, b):
  return pl.BlockSpec((a, b), lambda i: (0, 0))


def _tc_layer0(x, parts, d0, d1, Wl0, bl0, Wr0, Wl1, nsplit):
  dout = Wl0.shape[1]
  din = x.shape[1]
  nch = len(parts) // 2
  wc = din // nch
  dn = Wl1.shape[1]
  wn = dn // nsplit

  def body(*refs):
    x_r = refs[0]
    part_r = refs[1:1 + 2 * nch]
    d0_r, d1_r, Wl0_r, bl0_r, Wr0_r, Wl1_r = refs[1 + 2 * nch:7 + 2 * nch]
    outs = refs[7 + 2 * nch:]
    h1_o, invd_o = outs[0], outs[1]
    u_o = outs[2:]
    deg = jnp.maximum(d0_r[:, 0:1].astype(jnp.float32) +
                      d1_r[:, 0:1].astype(jnp.float32), 1.0)
    invd = 1.0 / deg
    invd_o[...] = invd
    mean = jnp.concatenate(
        [(part_r[2 * j][...].astype(jnp.float32) +
          part_r[2 * j + 1][...].astype(jnp.float32)) * invd
         for j in range(nch)], axis=1)
    h1 = jnp.tanh(jnp.dot(mean, Wl0_r[...], preferred_element_type=jnp.float32)
                  + bl0_r[0, :] +
                  jnp.dot(x_r[...], Wr0_r[...],
                          preferred_element_type=jnp.float32))
    h1_o[...] = h1
    u1 = jnp.dot(h1, Wl1_r[...], preferred_element_type=jnp.float32)
    for s in range(nsplit):
      u_o[s][...] = u1[:, s * wn:(s + 1) * wn].astype(_BF)

  in_specs = ([_row_spec(din)] + [_row_spec(wc)] * (2 * nch) +
              [_row_spec(16), _row_spec(16),
               _full_spec(din, dout), _full_spec(1, dout),
               _full_spec(din, dout), _full_spec(dout, dn)])
  out_specs = ([_row_spec(dout), _row_spec(1)] + [_row_spec(wn)] * nsplit)
  out_shape = ([jax.ShapeDtypeStruct((_N, dout), jnp.float32),
                jax.ShapeDtypeStruct((_N, 1), jnp.float32)] +
               [jax.ShapeDtypeStruct((_N, wn), _BF)] * nsplit)
  return pl.pallas_call(
      body, grid=(_N // _RB,),
      in_specs=in_specs, out_specs=out_specs, out_shape=out_shape,
  )(x, *parts, d0, d1, Wl0, bl0, Wr0, Wl1)


def _tc_layer_mid(h, parts, invd, bl, Wr, Wl_next, nsplit):
  din = h.shape[1]
  dout = bl.shape[1]
  nch = len(parts) // 2
  wc = dout // nch
  dn = Wl_next.shape[1]
  wn = dn // nsplit

  def body(*refs):
    h_r = refs[0]
    part_r = refs[1:1 + 2 * nch]
    invd_r, bl_r, Wr_r, Wln_r = refs[1 + 2 * nch:5 + 2 * nch]
    outs = refs[5 + 2 * nch:]
    hn_o = outs[0]
    u_o = outs[1:]
    invd = invd_r[...]
    mean = jnp.concatenate(
        [(part_r[2 * j][...].astype(jnp.float32) +
          part_r[2 * j + 1][...].astype(jnp.float32)) * invd
         for j in range(nch)], axis=1)
    hn = jnp.tanh(mean + bl_r[0, :] +
                  jnp.dot(h_r[...], Wr_r[...],
                          preferred_element_type=jnp.float32))
    hn_o[...] = hn
    un = jnp.dot(hn, Wln_r[...], preferred_element_type=jnp.float32)
    for s in range(nsplit):
      u_o[s][...] = un[:, s * wn:(s + 1) * wn].astype(_BF)

  in_specs = ([_row_spec(din)] + [_row_spec(wc)] * (2 * nch) +
              [_row_spec(1), _full_spec(1, dout),
               _full_spec(din, dout), _full_spec(dout, dn)])
  out_specs = [_row_spec(dout)] + [_row_spec(wn)] * nsplit
  out_shape = ([jax.ShapeDtypeStruct((_N, dout), jnp.float32)] +
               [jax.ShapeDtypeStruct((_N, wn), _BF)] * nsplit)
  return pl.pallas_call(
      body, grid=(_N // _RB,),
      in_specs=in_specs, out_specs=out_specs, out_shape=out_shape,
  )(h, *parts, invd, bl, Wr, Wl_next)


def _tc_layer_last(h, parts, invd, bl, Wr):
  din = h.shape[1]
  dout = bl.shape[1]

  def body(h_r, p0_r, p1_r, invd_r, bl_r, Wr_r, out_o):
    mean = (p0_r[...].astype(jnp.float32) +
            p1_r[...].astype(jnp.float32)) * invd_r[...]
    out_o[...] = jax.nn.sigmoid(
        mean + bl_r[0, :] +
        jnp.dot(h_r[...], Wr_r[...], preferred_element_type=jnp.float32))

  return pl.pallas_call(
      body, grid=(_N // _RB,),
      in_specs=[_row_spec(din), _row_spec(dout), _row_spec(dout),
                _row_spec(1), _full_spec(1, dout),
                _full_spec(din, dout)],
      out_specs=[_row_spec(dout)],
      out_shape=[jax.ShapeDtypeStruct((_N, dout), jnp.float32)],
  )(h, parts[0], parts[1], invd, bl, Wr)[0]


def kernel(x, edge_index, batch, Wl0, bl0, Wr0, Wl1, bl1, Wr1, Wl2, bl2, Wr2,
           Wl3, bl3, Wr3, Wl4, bl4, Wr4):
  bl0, bl1, bl2, bl3, bl4 = (b.reshape(1, -1)
                             for b in (bl0, bl1, bl2, bl3, bl4))
  src2 = edge_index[0].reshape(_E // _BLK, _BLK)
  dst2 = edge_index[1].reshape(_E // _BLK, _BLK)

  ones16 = jnp.ones((_BLK, 16), _BF)
  zeros16 = jnp.zeros((_N, 16), _BF)
  deg0, deg1 = _make_sc_deg()(dst2, ones16, zeros16)

  xb = x.astype(_BF)
  z64 = jnp.zeros((_N, 64), _BF)
  parts0 = list(_make_sc_agg(64, 2)(xb[:, :64], xb[:, 64:], src2, dst2, z64))
  outs0 = _tc_layer0(x, parts0, deg0, deg1, Wl0, bl0, Wr0, Wl1, 2)
  h1, invd = outs0[0], outs0[1]

  h = h1
  u_chunks = list(outs0[2:])
  mids = [(bl1, Wr1, Wl2), (bl2, Wr2, Wl3), (bl3, Wr3, Wl4)]
  nsplits = [3, 2, 1]
  for (bl, Wr, Wln), nsplit in zip(mids, nsplits):
    wc = u_chunks[0].shape[1]
    zc = jnp.zeros((_N, wc), _BF)
    parts = list(_make_sc_agg(wc, len(u_chunks))(*u_chunks, src2, dst2, zc))
    outs = _tc_layer_mid(h, parts, invd, bl, Wr, Wln, nsplit)
    h = outs[0]
    u_chunks = list(outs[1:])

  p0, p1 = _make_sc_agg(16, 1)(u_chunks[0], src2, dst2, zeros16)
  return _tc_layer_last(h, [p0, p1], invd, bl4, Wr4)


# speedup vs baseline: 11.5139x; 1.1520x over previous
import functools

import jax
import jax.numpy as jnp
from jax import lax
from jax.experimental import pallas as pl
from jax.experimental.pallas import tpu as pltpu
from jax.experimental.pallas import tpu_sc as plsc

_N = 10000
_E = 320000

_NC = 2
_NS = 16
_NW = _NC * _NS
_BLK = 100
_EPT = _E // _NW
_NBT = _EPT // _BLK
_RPT = _N // _NS
_K = 5
_NO2 = _NBT // (2 * _K)

_BF = jnp.bfloat16


def _mesh():
  return plsc.VectorSubcoreMesh(
      core_axis_name="c", subcore_axis_name="s",
      num_cores=_NC, num_subcores=_NS)


def _make_sc_agg(wc, nch):
  @functools.partial(
      pl.kernel,
      out_type=tuple(jax.ShapeDtypeStruct((_N, wc), _BF)
                     for _ in range(2 * nch)),
      mesh=_mesh(),
      scratch_types=[
          pltpu.VMEM((_NBT, _BLK), jnp.int32),
          pltpu.VMEM((_NBT, _BLK), jnp.int32),
          pltpu.VMEM((2, _K, _BLK, wc), _BF),
          pltpu.VMEM_SHARED((_N, wc), _BF),
          pltpu.SemaphoreType.DMA,
          pltpu.SemaphoreType.DMA,
      ],
      compiler_params=pltpu.CompilerParams(use_tc_tiling_on_sc=False))
  def k(*refs):
    tables = refs[:nch]
    src_h, dst_h, zeros_h = refs[nch:nch + 3]
    outs = refs[nch + 3:nch + 3 + 2 * nch]
    src_v, dst_v, rows_v, acc, gsem, ssem = refs[nch + 3 + 2 * nch:]
    co = lax.axis_index("c")
    sid = lax.axis_index("s")
    wid = co * _NS + sid
    rs = sid * _RPT

    pltpu.sync_copy(src_h.at[pl.ds(wid * _NBT, _NBT)], src_v)
    pltpu.sync_copy(dst_h.at[pl.ds(wid * _NBT, _NBT)], dst_v)

    def fire_gathers(table_h, half, b0):
      return [pltpu.async_copy(table_h.at[src_v.at[b0 + p]],
                               rows_v.at[half].at[p], gsem)
              for p in range(_K)]

    def fire_scatters(half, b0):
      return [pltpu.async_copy(rows_v.at[half].at[p],
                               acc.at[dst_v.at[b0 + p]], ssem, add=True)
              for p in range(_K)]

    def drain_gathers(table_h, half):
      for p in range(_K):
        pltpu.make_async_copy(table_h.at[pl.ds(0, _BLK)],
                              rows_v.at[half].at[p], gsem).wait()

    def drain_scatters(half, b0):
      for p in range(_K):
        pltpu.make_async_copy(rows_v.at[half].at[p],
                              acc.at[dst_v.at[b0 + p]], ssem).wait()

    for j in range(nch):
      table_h = tables[j]
      pltpu.sync_copy(zeros_h.at[pl.ds(rs, _RPT)], acc.at[pl.ds(rs, _RPT)])
      plsc.subcore_barrier()

      fire_gathers(table_h, 0, 0)

      def body(o2, carry):
        b0 = 2 * o2 * _K
        b1 = b0 + _K
        drain_gathers(table_h, 0)

        @pl.when(o2 > 0)
        def _():
          drain_scatters(1, b0 - _K)

        gd = fire_gathers(table_h, 1, b1)
        sd = fire_scatters(0, b0)
        for d in gd:
          d.wait()
        for d in sd:
          d.wait()

        @pl.when(o2 < _NO2 - 1)
        def _():
          fire_gathers(table_h, 0, b1 + _K)

        fire_scatters(1, b1)
        return carry

      lax.fori_loop(0, _NO2, body, 0)
      drain_scatters(1, _NBT - _K)
      plsc.subcore_barrier()

      @pl.when(co == 0)
      def _():
        pltpu.sync_copy(acc.at[pl.ds(rs, _RPT)],
                        outs[2 * j].at[pl.ds(rs, _RPT)])

      @pl.when(co == 1)
      def _():
        pltpu.sync_copy(acc.at[pl.ds(rs, _RPT)],
                        outs[2 * j + 1].at[pl.ds(rs, _RPT)])

  return k


def _make_sc_deg():
  wc = 16

  @functools.partial(
      pl.kernel,
      out_type=(jax.ShapeDtypeStruct((_N, wc), _BF),
                jax.ShapeDtypeStruct((_N, wc), _BF)),
      mesh=_mesh(),
      scratch_types=[
          pltpu.VMEM((_NBT, _BLK), jnp.int32),
          pltpu.VMEM((_BLK, wc), _BF),
          pltpu.VMEM_SHARED((_N, wc), _BF),
          pltpu.SemaphoreType.DMA,
      ],
      compiler_params=pltpu.CompilerParams(use_tc_tiling_on_sc=False))
  def k(dst_h, ones_h, zeros_h, out0, out1, dst_v, ones_v, acc, ssem):
    co = lax.axis_index("c")
    sid = lax.axis_index("s")
    wid = co * _NS + sid
    rs = sid * _RPT

    pltpu.sync_copy(zeros_h.at[pl.ds(rs, _RPT)], acc.at[pl.ds(rs, _RPT)])
    pltpu.sync_copy(ones_h, ones_v)
    pltpu.sync_copy(dst_h.at[pl.ds(wid * _NBT, _NBT)], dst_v)
    plsc.subcore_barrier()

    def body(o, carry):
      b0 = o * _K
      sd = [pltpu.async_copy(ones_v, acc.at[dst_v.at[b0 + p]], ssem, add=True)
            for p in range(_K)]
      for d in sd:
        d.wait()
      return carry

    lax.fori_loop(0, _NBT // _K, body, 0)
    plsc.subcore_barrier()

    @pl.when(co == 0)
    def _():
      pltpu.sync_copy(acc.at[pl.ds(rs, _RPT)], out0.at[pl.ds(rs, _RPT)])

    @pl.when(co == 1)
    def _():
      pltpu.sync_copy(acc.at[pl.ds(rs, _RPT)], out1.at[pl.ds(rs, _RPT)])

  return k


_RB = 400


def _row_spec(d):
  return pl.BlockSpec((_RB, d), lambda i: (i, 0))


def _full_spec(a, b):
  return pl.BlockSpec((a, b), lambda i: (0, 0))


def _tc_layer0(x, parts, d0, d1, Wl0, bl0, Wr0, Wl1, nsplit):
  dout = Wl0.shape[1]
  din = x.shape[1]
  nch = len(parts) // 2
  wc = din // nch
  dn = Wl1.shape[1]
  wn = dn // nsplit

  def body(*refs):
    x_r = refs[0]
    part_r = refs[1:1 + 2 * nch]
    d0_r, d1_r, Wl0_r, bl0_r, Wr0_r, Wl1_r = refs[1 + 2 * nch:7 + 2 * nch]
    outs = refs[7 + 2 * nch:]
    h1_o, invd_o = outs[0], outs[1]
    u_o = outs[2:]
    deg = jnp.maximum(d0_r[:, 0:1].astype(jnp.float32) +
                      d1_r[:, 0:1].astype(jnp.float32), 1.0)
    invd = 1.0 / deg
    invd_o[...] = invd
    mean = jnp.concatenate(
        [(part_r[2 * j][...].astype(jnp.float32) +
          part_r[2 * j + 1][...].astype(jnp.float32)) * invd
         for j in range(nch)], axis=1)
    h1 = jnp.tanh(jnp.dot(mean, Wl0_r[...], preferred_element_type=jnp.float32)
                  + bl0_r[0, :] +
                  jnp.dot(x_r[...], Wr0_r[...],
                          preferred_element_type=jnp.float32))
    h1_o[...] = h1
    u1 = jnp.dot(h1, Wl1_r[...], preferred_element_type=jnp.float32)
    for s in range(nsplit):
      u_o[s][...] = u1[:, s * wn:(s + 1) * wn].astype(_BF)

  in_specs = ([_row_spec(din)] + [_row_spec(wc)] * (2 * nch) +
              [_row_spec(16), _row_spec(16),
               _full_spec(din, dout), _full_spec(1, dout),
               _full_spec(din, dout), _full_spec(dout, dn)])
  out_specs = ([_row_spec(dout), _row_spec(1)] + [_row_spec(wn)] * nsplit)
  out_shape = ([jax.ShapeDtypeStruct((_N, dout), jnp.float32),
                jax.ShapeDtypeStruct((_N, 1), jnp.float32)] +
               [jax.ShapeDtypeStruct((_N, wn), _BF)] * nsplit)
  return pl.pallas_call(
      body, grid=(_N // _RB,),
      in_specs=in_specs, out_specs=out_specs, out_shape=out_shape,
  )(x, *parts, d0, d1, Wl0, bl0, Wr0, Wl1)


def _tc_layer_mid(h, parts, invd, bl, Wr, Wl_next, nsplit):
  din = h.shape[1]
  dout = bl.shape[1]
  nch = len(parts) // 2
  wc = dout // nch
  dn = Wl_next.shape[1]
  wn = dn // nsplit

  def body(*refs):
    h_r = refs[0]
    part_r = refs[1:1 + 2 * nch]
    invd_r, bl_r, Wr_r, Wln_r = refs[1 + 2 * nch:5 + 2 * nch]
    outs = refs[5 + 2 * nch:]
    hn_o = outs[0]
    u_o = outs[1:]
    invd = invd_r[...]
    mean = jnp.concatenate(
        [(part_r[2 * j][...].astype(jnp.float32) +
          part_r[2 * j + 1][...].astype(jnp.float32)) * invd
         for j in range(nch)], axis=1)
    hn = jnp.tanh(mean + bl_r[0, :] +
                  jnp.dot(h_r[...], Wr_r[...],
                          preferred_element_type=jnp.float32))
    hn_o[...] = hn
    un = jnp.dot(hn, Wln_r[...], preferred_element_type=jnp.float32)
    for s in range(nsplit):
      u_o[s][...] = un[:, s * wn:(s + 1) * wn].astype(_BF)

  in_specs = ([_row_spec(din)] + [_row_spec(wc)] * (2 * nch) +
              [_row_spec(1), _full_spec(1, dout),
               _full_spec(din, dout), _full_spec(dout, dn)])
  out_specs = [_row_spec(dout)] + [_row_spec(wn)] * nsplit
  out_shape = ([jax.ShapeDtypeStruct((_N, dout), jnp.float32)] +
               [jax.ShapeDtypeStruct((_N, wn), _BF)] * nsplit)
  return pl.pallas_call(
      body, grid=(_N // _RB,),
      in_specs=in_specs, out_specs=out_specs, out_shape=out_shape,
  )(h, *parts, invd, bl, Wr, Wl_next)


def _tc_layer_last(h, parts, invd, bl, Wr):
  din = h.shape[1]
  dout = bl.shape[1]

  def body(h_r, p0_r, p1_r, invd_r, bl_r, Wr_r, out_o):
    mean = (p0_r[...].astype(jnp.float32) +
            p1_r[...].astype(jnp.float32)) * invd_r[...]
    out_o[...] = jax.nn.sigmoid(
        mean + bl_r[0, :] +
        jnp.dot(h_r[...], Wr_r[...], preferred_element_type=jnp.float32))

  return pl.pallas_call(
      body, grid=(_N // _RB,),
      in_specs=[_row_spec(din), _row_spec(dout), _row_spec(dout),
                _row_spec(1), _full_spec(1, dout),
                _full_spec(din, dout)],
      out_specs=[_row_spec(dout)],
      out_shape=[jax.ShapeDtypeStruct((_N, dout), jnp.float32)],
  )(h, parts[0], parts[1], invd, bl, Wr)[0]


def kernel(x, edge_index, batch, Wl0, bl0, Wr0, Wl1, bl1, Wr1, Wl2, bl2, Wr2,
           Wl3, bl3, Wr3, Wl4, bl4, Wr4):
  bl0, bl1, bl2, bl3, bl4 = (b.reshape(1, -1)
                             for b in (bl0, bl1, bl2, bl3, bl4))
  src2 = edge_index[0].reshape(_E // _BLK, _BLK)
  dst2 = edge_index[1].reshape(_E // _BLK, _BLK)

  ones16 = jnp.ones((_BLK, 16), _BF)
  zeros16 = jnp.zeros((_N, 16), _BF)
  deg0, deg1 = _make_sc_deg()(dst2, ones16, zeros16)

  xb = x.astype(_BF)
  z64 = jnp.zeros((_N, 64), _BF)
  parts0 = list(_make_sc_agg(64, 2)(xb[:, :64], xb[:, 64:], src2, dst2, z64))
  outs0 = _tc_layer0(x, parts0, deg0, deg1, Wl0, bl0, Wr0, Wl1, 2)
  h1, invd = outs0[0], outs0[1]

  h = h1
  u_chunks = list(outs0[2:])
  mids = [(bl1, Wr1, Wl2), (bl2, Wr2, Wl3), (bl3, Wr3, Wl4)]
  nsplits = [3, 2, 1]
  for (bl, Wr, Wln), nsplit in zip(mids, nsplits):
    wc = u_chunks[0].shape[1]
    zc = jnp.zeros((_N, wc), _BF)
    parts = list(_make_sc_agg(wc, len(u_chunks))(*u_chunks, src2, dst2, zc))
    outs = _tc_layer_mid(h, parts, invd, bl, Wr, Wln, nsplit)
    h = outs[0]
    u_chunks = list(outs[1:])

  p0, p1 = _make_sc_agg(16, 1)(u_chunks[0], src2, dst2, zeros16)
  return _tc_layer_last(h, [p0, p1], invd, bl4, Wr4)
